# Initial kernel scaffold; baseline (speedup 1.0000x reference)
import jax
import jax.numpy as jnp
from jax.experimental import pallas as pl


def kernel(queries, keys):
    raise NotImplementedError("write your pallas kernel here")


# speedup vs baseline: 32.6255x; 32.6255x over previous
import functools

import jax
import jax.numpy as jnp
from jax import lax
from jax.experimental import pallas as pl
from jax.experimental.pallas import tpu as pltpu
from jax.experimental.pallas import tpu_sc as plsc

Q = 1024
D = 128
M = 1000000
K_TOP = 100
QT = 256
CM = 2048
B = 128
MP = 489 * CM
NBLK = MP // B
NEG = -3.0e38
BIG = 1 << 20

NW = 32
QPW = Q // NW

N1 = 7936
NC1 = 496
L1P1 = 512
NS1 = 32

N2 = 12800
NC2 = 800
L1P2 = 1024
NS2 = 64


def _mm_kernel(q_ref, k_ref, s_ref, a_ref):
    j = pl.program_id(1)
    s = jax.lax.dot_general(
        q_ref[...], k_ref[...], (((1,), (1,)), ((), ())),
        preferred_element_type=jnp.float32)
    col = jax.lax.broadcasted_iota(jnp.int32, (QT, CM), 1) + j * CM
    s = jnp.where(col < M, s, NEG)
    s_ref[...] = s
    a_ref[0, :, :] = jnp.max(s.reshape(QT, CM // B, B), axis=-1)


def _scores_and_blockmax(queries, keys):
    grid = (Q // QT, MP // CM)
    return pl.pallas_call(
        _mm_kernel,
        grid=grid,
        in_specs=[
            pl.BlockSpec((QT, D), lambda i, j: (i, 0)),
            pl.BlockSpec((CM, D), lambda i, j: (j, 0)),
        ],
        out_specs=[
            pl.BlockSpec((QT, CM), lambda i, j: (i, j)),
            pl.BlockSpec((1, QT, CM // B), lambda i, j: (j, i, 0)),
        ],
        out_shape=[
            jax.ShapeDtypeStruct((Q, MP), jnp.float32),
            jax.ShapeDtypeStruct((MP // CM, Q, CM // B), jnp.float32),
        ],
    )(queries, keys)


_GDN = lax.GatherDimensionNumbers(
    offset_dims=(), collapsed_slice_dims=(0,), start_index_map=(0,))


def _perm(v, idx):
    return lax.gather(v, idx[:, None], _GDN, (1,),
                      mode=lax.GatherScatterMode.PROMISE_IN_BOUNDS)


def _bfly(v, iota, op):
    for k in (8, 4, 2, 1):
        v = op(v, _perm(v, iota ^ k))
    return v


def _build_l1(load_vec, c_ref, n_entries, stride):
    def body(b, _):
        base = b * 16
        acc = load_vec(base)
        for j in range(1, 16):
            acc = jnp.maximum(acc, load_vec(j * stride + base))
        c_ref[pl.ds(base, 16)] = acc
        return 0
    lax.fori_loop(0, n_entries // 16, body, 0)


def _locate(l2_ref, n_l2, iota):
    nv = n_l2 // 16
    vs = [l2_ref[pl.ds(i * 16, 16)] for i in range(nv)]
    mx = vs[0]
    for v in vs[1:]:
        mx = jnp.maximum(mx, v)
    g = _bfly(mx, iota, jnp.maximum)
    idx = jnp.where(vs[0] == g, iota, BIG)
    for i in range(1, nv):
        idx = jnp.minimum(idx, jnp.where(vs[i] == g, iota + 16 * i, BIG))
    e2 = _bfly(idx, iota, jnp.minimum)
    return g, e2


def _first_lane(v, g, iota):
    return _bfly(jnp.where(v == g, iota, BIG), iota, jnp.minimum)


def _scat1(ref, pos, val, iota):
    plsc.store_scatter(ref, [iota * 0 + pos],
                       jnp.zeros((16,), ref.dtype) + val, mask=iota == 0)


def _sc_topk(a2, s2):
    mesh = plsc.VectorSubcoreMesh(core_axis_name="c", subcore_axis_name="s")

    @functools.partial(
        pl.kernel, mesh=mesh,
        compiler_params=pltpu.CompilerParams(
            needs_layout_passes=False, use_tc_tiling_on_sc=False),
        out_type=[
            jax.ShapeDtypeStruct((Q, 128), jnp.float32),
            jax.ShapeDtypeStruct((Q, 128), jnp.int32),
        ],
        scratch_types=[
            pltpu.VMEM((N1,), jnp.float32),
            pltpu.VMEM((L1P1,), jnp.float32),
            pltpu.VMEM((NS1,), jnp.float32),
            pltpu.VMEM((K_TOP, 128), jnp.float32),
            pltpu.VMEM((L1P2,), jnp.float32),
            pltpu.VMEM((NS2,), jnp.float32),
            pltpu.VMEM((K_TOP,), jnp.int32),
            pltpu.VMEM((128,), jnp.float32),
            pltpu.VMEM((128,), jnp.int32),
            pltpu.SemaphoreType.DMA,
        ])
    def sck(a_hbm, s_hbm, vals_hbm, idx_hbm,
            x1, c1, l2a, rows, c2, l2b, rowidx, outv, outi, sem):
        iota = lax.broadcasted_iota(jnp.int32, (16,), 0)
        negv = jnp.full((16,), NEG, jnp.float32)
        wid = lax.axis_index("s") * 2 + lax.axis_index("c")

        for i in range((N1 - NBLK) // 16):
            x1[pl.ds(NBLK + i * 16, 16)] = negv
        for i in range((L1P1 - NC1) // 16):
            c1[pl.ds(NC1 + i * 16, 16)] = negv
        for i in range((L1P2 - NC2) // 16):
            c2[pl.ds(NC2 + i * 16, 16)] = negv

        def per_query(t, _):
            q = wid * QPW + t
            pltpu.sync_copy(a_hbm.at[q], x1.at[pl.ds(0, NBLK)])

            _build_l1(lambda off: x1[pl.ds(off, 16)], c1, NC1, NC1)
            _build_l1(lambda off: c1[pl.ds(off, 16)], l2a, NS1, NS1)

            qbase = q * NBLK

            def ext1(t2, _):
                g, e2 = _locate(l2a, NS1, iota)
                v_l1 = plsc.load_gather(c1, [e2 + iota * NS1])
                j1 = _first_lane(v_l1, g, iota)
                e1 = e2 + j1 * NS1
                vx = plsc.load_gather(x1, [e1 + iota * NC1])
                j2 = _first_lane(vx, g, iota)
                p = e1 + j2 * NC1
                _scat1(rowidx, t2, qbase + p, iota)
                _scat1(x1, p, NEG, iota)
                m1 = _bfly(jnp.where(iota == j2, negv, vx), iota, jnp.maximum)
                _scat1(c1, e1, m1, iota)
                m2 = _bfly(jnp.where(iota == j1, m1, v_l1), iota, jnp.maximum)
                _scat1(l2a, e2, m2, iota)
                return 0

            lax.fori_loop(0, K_TOP, ext1, 0)

            pltpu.async_copy(s_hbm.at[rowidx], rows, sem).wait()

            def load2(off):
                fl = off + iota
                return plsc.load_gather(rows, [fl >> 7, fl & 127])

            _build_l1(load2, c2, NC2, NC2)
            _build_l1(lambda off: c2[pl.ds(off, 16)], l2b, NS2, NS2)

            def ext2(t2, _):
                g, e2 = _locate(l2b, NS2, iota)
                v_l1 = plsc.load_gather(c2, [e2 + iota * NS2])
                j1 = _first_lane(v_l1, g, iota)
                e1 = e2 + j1 * NS2
                plane = e1 + iota * NC2
                vx = plsc.load_gather(rows, [plane >> 7, plane & 127])
                j2 = _first_lane(vx, g, iota)
                p = e1 + j2 * NC2
                r = p >> 7
                col = p & 127
                bid = plsc.load_gather(rowidx, [r]) - qbase
                gi = bid * B + col
                _scat1(outv, t2, g, iota)
                plsc.store_scatter(outi, [iota * 0 + t2], gi, mask=iota == 0)
                plsc.store_scatter(rows, [r, col], negv, mask=iota == 0)
                m1 = _bfly(jnp.where(iota == j2, negv, vx), iota, jnp.maximum)
                _scat1(c2, e1, m1, iota)
                m2 = _bfly(jnp.where(iota == j1, m1, v_l1), iota, jnp.maximum)
                _scat1(l2b, e2, m2, iota)
                return 0

            lax.fori_loop(0, K_TOP, ext2, 0)

            pltpu.sync_copy(outv, vals_hbm.at[q])
            pltpu.sync_copy(outi, idx_hbm.at[q])
            return 0

        lax.fori_loop(0, QPW, per_query, 0)

    return sck(a2, s2)


def kernel(queries, keys):
    s, a3 = _scores_and_blockmax(queries, keys)
    a2 = a3.transpose(1, 0, 2).reshape(Q, NBLK)
    s2 = s.reshape(Q * NBLK, B)
    vals_p, idx_p = _sc_topk(a2, s2)
    return (vals_p[:, :K_TOP], idx_p[:, :K_TOP])


# speedup vs baseline: 39.9075x; 1.2232x over previous
import functools

import jax
import jax.numpy as jnp
from jax import lax
from jax.experimental import pallas as pl
from jax.experimental.pallas import tpu as pltpu
from jax.experimental.pallas import tpu_sc as plsc

Q = 1024
D = 128
M = 1000000
K_TOP = 100
QT = 256
CM = 16384
B = 128
MP = 62 * CM
NBLK = MP // B
NEG = -3.0e38
BIG = 1 << 20

NW = 32
QPW = Q // NW

N1 = NBLK
NC1 = 496
L1P1 = 512
NS1 = 32

N2 = 12800
NC2 = 800
L1P2 = 1024
NS2 = 64


def _mm_kernel(q_ref, k_ref, s_ref, a_ref):
    j = pl.program_id(1)
    s = jax.lax.dot_general(
        q_ref[...], k_ref[...], (((1,), (1,)), ((), ())),
        preferred_element_type=jnp.float32)
    col = jax.lax.broadcasted_iota(jnp.int32, (QT, CM), 1) + j * CM
    s = jnp.where(col < M, s, NEG)
    s_ref[...] = s
    a_ref[...] = jnp.max(s.reshape(QT, CM // B, B), axis=-1)


def _scores_and_blockmax(queries, keys):
    grid = (Q // QT, MP // CM)
    return pl.pallas_call(
        _mm_kernel,
        grid=grid,
        in_specs=[
            pl.BlockSpec((QT, D), lambda i, j: (i, 0)),
            pl.BlockSpec((CM, D), lambda i, j: (j, 0)),
        ],
        out_specs=[
            pl.BlockSpec((QT, CM), lambda i, j: (i, j)),
            pl.BlockSpec((QT, CM // B), lambda i, j: (i, j)),
        ],
        out_shape=[
            jax.ShapeDtypeStruct((Q, MP), jnp.float32),
            jax.ShapeDtypeStruct((Q, NBLK), jnp.float32),
        ],
    )(queries, keys)


_GDN = lax.GatherDimensionNumbers(
    offset_dims=(), collapsed_slice_dims=(0,), start_index_map=(0,))


def _perm(v, idx):
    return lax.gather(v, idx[:, None], _GDN, (1,),
                      mode=lax.GatherScatterMode.PROMISE_IN_BOUNDS)


def _bfly(v, iota, op):
    for k in (8, 4, 2, 1):
        v = op(v, _perm(v, iota ^ k))
    return v


def _build_l1(load_vec, c_ref, n_entries, stride):
    def body(b, _):
        base = b * 16
        acc = load_vec(base)
        for j in range(1, 16):
            acc = jnp.maximum(acc, load_vec(j * stride + base))
        c_ref[pl.ds(base, 16)] = acc
        return 0
    lax.fori_loop(0, n_entries // 16, body, 0)


def _locate(l2_ref, n_l2, iota):
    nv = n_l2 // 16
    vs = [l2_ref[pl.ds(i * 16, 16)] for i in range(nv)]
    mx = vs[0]
    for v in vs[1:]:
        mx = jnp.maximum(mx, v)
    g = _bfly(mx, iota, jnp.maximum)
    idx = jnp.where(vs[0] == g, iota, BIG)
    for i in range(1, nv):
        idx = jnp.minimum(idx, jnp.where(vs[i] == g, iota + 16 * i, BIG))
    e2 = _bfly(idx, iota, jnp.minimum)
    return g, e2


def _first_lane(v, g, iota):
    return _bfly(jnp.where(v == g, iota, BIG), iota, jnp.minimum)


def _scat1(ref, pos, val, iota):
    plsc.store_scatter(ref, [iota * 0 + pos],
                       jnp.zeros((16,), ref.dtype) + val, mask=iota == 0)


def _sc_topk(a2, s2):
    mesh = plsc.VectorSubcoreMesh(core_axis_name="c", subcore_axis_name="s")

    @functools.partial(
        pl.kernel, mesh=mesh,
        compiler_params=pltpu.CompilerParams(
            needs_layout_passes=False, use_tc_tiling_on_sc=False),
        out_type=[
            jax.ShapeDtypeStruct((Q, 128), jnp.float32),
            jax.ShapeDtypeStruct((Q, 128), jnp.int32),
        ],
        scratch_types=[
            pltpu.VMEM((N1,), jnp.float32),
            pltpu.VMEM((L1P1,), jnp.float32),
            pltpu.VMEM((NS1,), jnp.float32),
            pltpu.VMEM((K_TOP, 128), jnp.float32),
            pltpu.VMEM((L1P2,), jnp.float32),
            pltpu.VMEM((NS2,), jnp.float32),
            pltpu.VMEM((K_TOP,), jnp.int32),
            pltpu.VMEM((128,), jnp.float32),
            pltpu.VMEM((128,), jnp.int32),
            pltpu.SemaphoreType.DMA,
        ])
    def sck(a_hbm, s_hbm, vals_hbm, idx_hbm,
            x1, c1, l2a, rows, c2, l2b, rowidx, outv, outi, sem):
        iota = lax.broadcasted_iota(jnp.int32, (16,), 0)
        negv = jnp.full((16,), NEG, jnp.float32)
        wid = lax.axis_index("s") * 2 + lax.axis_index("c")

        for i in range((L1P1 - NC1) // 16):
            c1[pl.ds(NC1 + i * 16, 16)] = negv
        for i in range((L1P2 - NC2) // 16):
            c2[pl.ds(NC2 + i * 16, 16)] = negv

        def per_query(t, _):
            q = wid * QPW + t
            pltpu.sync_copy(a_hbm.at[q], x1)

            _build_l1(lambda off: x1[pl.ds(off, 16)], c1, NC1, NC1)
            _build_l1(lambda off: c1[pl.ds(off, 16)], l2a, NS1, NS1)

            qbase = q * NBLK

            def ext1(t2, _):
                g, e2 = _locate(l2a, NS1, iota)
                v_l1 = plsc.load_gather(c1, [e2 + iota * NS1])
                j1 = _first_lane(v_l1, g, iota)
                e1 = e2 + j1 * NS1
                vx = plsc.load_gather(x1, [e1 + iota * NC1])
                j2 = _first_lane(vx, g, iota)
                p = e1 + j2 * NC1
                _scat1(rowidx, t2, qbase + p, iota)
                _scat1(x1, p, NEG, iota)
                m1 = _bfly(jnp.where(iota == j2, negv, vx), iota, jnp.maximum)
                _scat1(c1, e1, m1, iota)
                m2 = _bfly(jnp.where(iota == j1, m1, v_l1), iota, jnp.maximum)
                _scat1(l2a, e2, m2, iota)
                return 0

            lax.fori_loop(0, K_TOP, ext1, 0)

            pltpu.async_copy(s_hbm.at[rowidx], rows, sem).wait()

            def load2(off):
                fl = off + iota
                return plsc.load_gather(rows, [fl >> 7, fl & 127])

            _build_l1(load2, c2, NC2, NC2)
            _build_l1(lambda off: c2[pl.ds(off, 16)], l2b, NS2, NS2)

            def ext2(t2, _):
                g, e2 = _locate(l2b, NS2, iota)
                v_l1 = plsc.load_gather(c2, [e2 + iota * NS2])
                j1 = _first_lane(v_l1, g, iota)
                e1 = e2 + j1 * NS2
                plane = e1 + iota * NC2
                vx = plsc.load_gather(rows, [plane >> 7, plane & 127])
                j2 = _first_lane(vx, g, iota)
                p = e1 + j2 * NC2
                r = p >> 7
                col = p & 127
                bid = plsc.load_gather(rowidx, [r]) - qbase
                gi = bid * B + col
                _scat1(outv, t2, g, iota)
                plsc.store_scatter(outi, [iota * 0 + t2], gi, mask=iota == 0)
                plsc.store_scatter(rows, [r, col], negv, mask=iota == 0)
                m1 = _bfly(jnp.where(iota == j2, negv, vx), iota, jnp.maximum)
                _scat1(c2, e1, m1, iota)
                m2 = _bfly(jnp.where(iota == j1, m1, v_l1), iota, jnp.maximum)
                _scat1(l2b, e2, m2, iota)
                return 0

            lax.fori_loop(0, K_TOP, ext2, 0)

            pltpu.sync_copy(outv, vals_hbm.at[q])
            pltpu.sync_copy(outi, idx_hbm.at[q])
            return 0

        lax.fori_loop(0, QPW, per_query, 0)

    return sck(a2, s2)


def kernel(queries, keys):
    s, a2 = _scores_and_blockmax(queries, keys)
    s2 = s.reshape(Q * NBLK, B)
    vals_p, idx_p = _sc_topk(a2, s2)
    return (vals_p[:, :K_TOP], idx_p[:, :K_TOP])


# speedup vs baseline: 77.9572x; 1.9534x over previous
import functools

import jax
import jax.numpy as jnp
from jax import lax
from jax.experimental import pallas as pl
from jax.experimental.pallas import tpu as pltpu
from jax.experimental.pallas import tpu_sc as plsc

Q = 1024
D = 128
M = 1000000
K_TOP = 100
QT = 256
CM = 16384
B = 128
MP = 62 * CM
NBLK = MP // B
NEG = -3.0e38
BIG = 1 << 20

NW = 32
QPW = Q // NW

N1 = NBLK
NC1 = 496
L1P1 = 512
NS1 = 32

N2 = 12800
NC2 = 800
L1P2 = 1024
NS2 = 64


def _mm_kernel(q_ref, k_ref, s_ref, a_ref):
    j = pl.program_id(1)
    s = jax.lax.dot_general(
        q_ref[...], k_ref[...], (((1,), (1,)), ((), ())),
        preferred_element_type=jnp.float32)
    col = jax.lax.broadcasted_iota(jnp.int32, (QT, CM), 1) + j * CM
    s = jnp.where(col < M, s, NEG)
    s_ref[...] = s
    a_ref[...] = jnp.max(s.reshape(QT, CM // B, B), axis=-1)


def _scores_and_blockmax(queries, keys):
    grid = (Q // QT, MP // CM)
    return pl.pallas_call(
        _mm_kernel,
        grid=grid,
        in_specs=[
            pl.BlockSpec((QT, D), lambda i, j: (i, 0)),
            pl.BlockSpec((CM, D), lambda i, j: (j, 0)),
        ],
        out_specs=[
            pl.BlockSpec((QT, CM), lambda i, j: (i, j)),
            pl.BlockSpec((QT, CM // B), lambda i, j: (i, j)),
        ],
        out_shape=[
            jax.ShapeDtypeStruct((Q, MP), jnp.float32),
            jax.ShapeDtypeStruct((Q, NBLK), jnp.float32),
        ],
    )(queries, keys)


_GDN = lax.GatherDimensionNumbers(
    offset_dims=(), collapsed_slice_dims=(0,), start_index_map=(0,))


def _perm(v, idx):
    return lax.gather(v, idx[:, None], _GDN, (1,),
                      mode=lax.GatherScatterMode.PROMISE_IN_BOUNDS)


def _bfly(v, iota, op):
    for k in (8, 4, 2, 1):
        v = op(v, _perm(v, iota ^ k))
    return v


def _build_l1(load_vec, c_ref, n_entries, stride):
    def body(b, _):
        base = b * 16
        acc = load_vec(base)
        for j in range(1, 16):
            acc = jnp.maximum(acc, load_vec(j * stride + base))
        c_ref[pl.ds(base, 16)] = acc
        return 0
    lax.fori_loop(0, n_entries // 16, body, 0)


def _locate(l2_ref, n_l2, iota):
    nv = n_l2 // 16
    vs = [l2_ref[pl.ds(i * 16, 16)] for i in range(nv)]
    mx = vs[0]
    for v in vs[1:]:
        mx = jnp.maximum(mx, v)
    g = _bfly(mx, iota, jnp.maximum)
    idx = jnp.where(vs[0] == g, iota, BIG)
    for i in range(1, nv):
        idx = jnp.minimum(idx, jnp.where(vs[i] == g, iota + 16 * i, BIG))
    e2 = _bfly(idx, iota, jnp.minimum)
    return g, e2


def _first_lane(v, g, iota):
    return _bfly(jnp.where(v == g, iota, BIG), iota, jnp.minimum)


def _scat1(ref, pos, val, iota):
    plsc.store_scatter(ref, [iota * 0 + pos],
                       jnp.zeros((16,), ref.dtype) + val, mask=iota == 0)


def _sc_stage1(a2):
    mesh = plsc.VectorSubcoreMesh(core_axis_name="c", subcore_axis_name="s")

    @functools.partial(
        pl.kernel, mesh=mesh,
        compiler_params=pltpu.CompilerParams(
            needs_layout_passes=False, use_tc_tiling_on_sc=False),
        out_type=[
            jax.ShapeDtypeStruct((Q, 128), jnp.int32),
        ],
        scratch_types=[
            pltpu.VMEM((N1,), jnp.float32),
            pltpu.VMEM((L1P1,), jnp.float32),
            pltpu.VMEM((NS1,), jnp.float32),
            pltpu.VMEM((128,), jnp.int32),
            pltpu.SemaphoreType.DMA,
        ])
    def sck(a_hbm, bids_hbm, x1, c1, l2a, outb, sem):
        iota = lax.broadcasted_iota(jnp.int32, (16,), 0)
        negv = jnp.full((16,), NEG, jnp.float32)
        zerov = jnp.zeros((16,), jnp.int32)
        wid = lax.axis_index("s") * 2 + lax.axis_index("c")

        for i in range((L1P1 - NC1) // 16):
            c1[pl.ds(NC1 + i * 16, 16)] = negv
        outb[pl.ds(96, 16)] = zerov
        outb[pl.ds(112, 16)] = zerov

        def per_query(t, _):
            q = wid * QPW + t
            pltpu.sync_copy(a_hbm.at[q], x1)

            _build_l1(lambda off: x1[pl.ds(off, 16)], c1, NC1, NC1)
            _build_l1(lambda off: c1[pl.ds(off, 16)], l2a, NS1, NS1)

            def ext1(t2, _):
                g, e2 = _locate(l2a, NS1, iota)
                v_l1 = plsc.load_gather(c1, [e2 + iota * NS1])
                j1 = _first_lane(v_l1, g, iota)
                e1 = e2 + j1 * NS1
                vx = plsc.load_gather(x1, [e1 + iota * NC1])
                j2 = _first_lane(vx, g, iota)
                p = e1 + j2 * NC1
                _scat1(outb, t2, p, iota)
                _scat1(x1, p, NEG, iota)
                m1 = _bfly(jnp.where(iota == j2, negv, vx), iota, jnp.maximum)
                _scat1(c1, e1, m1, iota)
                m2 = _bfly(jnp.where(iota == j1, m1, v_l1), iota, jnp.maximum)
                _scat1(l2a, e2, m2, iota)
                return 0

            lax.fori_loop(0, K_TOP, ext1, 0)
            pltpu.sync_copy(outb, bids_hbm.at[q])
            return 0

        lax.fori_loop(0, QPW, per_query, 0)

    return sck(a2)


def _sc_stage2(s, bids):
    mesh = plsc.VectorSubcoreMesh(core_axis_name="c", subcore_axis_name="s")

    @functools.partial(
        pl.kernel, mesh=mesh,
        compiler_params=pltpu.CompilerParams(
            needs_layout_passes=False, use_tc_tiling_on_sc=True),
        out_type=[
            jax.ShapeDtypeStruct((Q, 128), jnp.float32),
            jax.ShapeDtypeStruct((Q, 128), jnp.int32),
        ],
        scratch_types=[
            pltpu.VMEM((128,), jnp.int32),
            pltpu.VMEM((112 * 128,), jnp.float32),
            pltpu.VMEM((L1P2,), jnp.float32),
            pltpu.VMEM((NS2,), jnp.float32),
            pltpu.VMEM((128,), jnp.float32),
            pltpu.VMEM((128,), jnp.int32),
            pltpu.SemaphoreType.DMA,
            pltpu.SemaphoreType.DMA,
        ])
    def sck(s_hbm, bids_hbm, vals_hbm, idx_hbm,
            bidv, rows, c2, l2b, outv, outi, semg, sem):
        iota = lax.broadcasted_iota(jnp.int32, (16,), 0)
        negv = jnp.full((16,), NEG, jnp.float32)
        wid = lax.axis_index("s") * 2 + lax.axis_index("c")

        for i in range((L1P2 - NC2) // 16):
            c2[pl.ds(NC2 + i * 16, 16)] = negv

        def per_query(t, _):
            q = wid * QPW + t
            pltpu.sync_copy(bids_hbm.at[q], bidv)

            def fire16(g, _):
                bv = bidv[pl.ds(g * 16, 16)]
                handles = []
                for i in range(16):
                    bt = bv[i]
                    h = pltpu.async_copy(
                        s_hbm.at[q, pl.ds(bt * B, B)],
                        rows.at[pl.ds((g * 16 + i) * B, B)], sem)
                    handles.append(h)
                for h in handles:
                    h.wait()
                return 0

            lax.fori_loop(0, 7, fire16, 0)

            _build_l1(lambda off: rows[pl.ds(off, 16)], c2, NC2, NC2)
            _build_l1(lambda off: c2[pl.ds(off, 16)], l2b, NS2, NS2)

            def ext2(t2, _):
                g, e2 = _locate(l2b, NS2, iota)
                v_l1 = plsc.load_gather(c2, [e2 + iota * NS2])
                j1 = _first_lane(v_l1, g, iota)
                e1 = e2 + j1 * NS2
                vx = plsc.load_gather(rows, [e1 + iota * NC2])
                j2 = _first_lane(vx, g, iota)
                p = e1 + j2 * NC2
                r = p >> 7
                col = p & 127
                bid = plsc.load_gather(bidv, [r])
                gi = bid * B + col
                _scat1(outv, t2, g, iota)
                plsc.store_scatter(outi, [iota * 0 + t2], gi, mask=iota == 0)
                _scat1(rows, p, NEG, iota)
                m1 = _bfly(jnp.where(iota == j2, negv, vx), iota, jnp.maximum)
                _scat1(c2, e1, m1, iota)
                m2 = _bfly(jnp.where(iota == j1, m1, v_l1), iota, jnp.maximum)
                _scat1(l2b, e2, m2, iota)
                return 0

            lax.fori_loop(0, K_TOP, ext2, 0)

            pltpu.sync_copy(outv, vals_hbm.at[q])
            pltpu.sync_copy(outi, idx_hbm.at[q])
            return 0

        lax.fori_loop(0, QPW, per_query, 0)

    return sck(s, bids)


def kernel(queries, keys):
    s, a2 = _scores_and_blockmax(queries, keys)
    (bids,) = _sc_stage1(a2)
    vals_p, idx_p = _sc_stage2(s, bids)
    return (vals_p[:, :K_TOP], idx_p[:, :K_TOP])
